# baseline (device time: 1400133 ns/iter reference)
import jax
import jax.numpy as jnp
from jax import lax
from jax.experimental import pallas as pl
from jax.experimental.pallas import tpu as pltpu

N_DEV = 8
B = 2
S_PER = 512
N_OUT = 8192
K = 1024
TILE_M = 128
N_MTILES = S_PER // TILE_M
HOPS = N_DEV - 1


def _fused_matmul_reduce_scatter(Ob, Wb):

    def body(o_ref, w_ref, out_ref, recv_hbm, w_v, acc_v, vb, o_v, va, vf32,
             send_sems, recv_sems, o_sems, va_sem, st_sem):
        my = lax.axis_index("i")
        right = lax.rem(my + 1, N_DEV)

        def hop_rdma(s, b):
            return pltpu.make_async_remote_copy(
                src_ref=acc_v.at[b],
                dst_ref=recv_hbm.at[s, b],
                send_sem=send_sems.at[s * B + b],
                recv_sem=recv_sems.at[s * B + b],
                device_id=(right,),
                device_id_type=pl.DeviceIdType.MESH,
            )

        def load_o(c):
            for b in range(B):
                pltpu.make_async_copy(
                    o_ref.at[b, c], o_v.at[b], o_sems.at[b]).start()

        def compute_half(c, dst, b):
            pltpu.make_async_copy(
                o_ref.at[b, c], o_v.at[b], o_sems.at[b]).wait()

            def dot_tile(r, _):
                sl = pl.ds(r * TILE_M, TILE_M)
                dst[b, sl] = jnp.dot(
                    o_v[b, sl].astype(jnp.bfloat16), w_v[...],
                    preferred_element_type=jnp.float32,
                ).astype(jnp.bfloat16)
                return 0

            lax.fori_loop(0, N_MTILES, dot_tile, 0)

        def compute_chunk(c, dst):
            for b in range(B):
                compute_half(c, dst, b)

        pltpu.make_async_copy(w_ref, w_v, va_sem).start()
        c0 = lax.rem(my + N_DEV - 1, N_DEV)
        load_o(c0)
        pltpu.make_async_copy(w_ref, w_v, va_sem).wait()
        for b in range(B):
            compute_half(c0, acc_v, b)
            hop_rdma(0, b).start()

        def hop(s, _):
            c = lax.rem(my + 2 * N_DEV - 2 - s, N_DEV)
            final = s == HOPS - 1
            load_o(c)
            compute_chunk(c, vb)

            for b in range(B):
                d = hop_rdma(s, b)
                d.wait_recv()
                d.wait_send()

                def add_tile(t, _):
                    sl = pl.ds(t * TILE_M, TILE_M)
                    pltpu.make_async_copy(
                        recv_hbm.at[s, b, sl], va, va_sem).start()
                    pltpu.make_async_copy(
                        recv_hbm.at[s, b, sl], va, va_sem).wait()

                    @pl.when(jnp.logical_not(final))
                    def _():
                        acc_v[b, sl] = va[...] + vb[b, sl]

                    @pl.when(final)
                    def _():
                        vf32[...] = (va[...].astype(jnp.float32)
                                     + vb[b, sl].astype(jnp.float32))
                        pltpu.make_async_copy(
                            vf32, out_ref.at[b, sl], st_sem).start()
                        pltpu.make_async_copy(
                            vf32, out_ref.at[b, sl], st_sem).wait()

                    return 0

                lax.fori_loop(0, N_MTILES, add_tile, 0)

                @pl.when(jnp.logical_not(final))
                def _():
                    hop_rdma(s + 1, b).start()
            return 0

        lax.fori_loop(0, HOPS, hop, 0)

    hbm = pltpu.MemorySpace.HBM
    out, _ = pl.pallas_call(
        body,
        out_shape=(
            jax.ShapeDtypeStruct((B, S_PER, N_OUT), jnp.float32),
            jax.ShapeDtypeStruct((HOPS, B, S_PER, N_OUT), jnp.bfloat16),
        ),
        in_specs=[
            pl.BlockSpec(memory_space=hbm),
            pl.BlockSpec(memory_space=hbm),
        ],
        out_specs=(
            pl.BlockSpec(memory_space=hbm),
            pl.BlockSpec(memory_space=hbm),
        ),
        scratch_shapes=[
            pltpu.VMEM((K, N_OUT), jnp.bfloat16),
            pltpu.VMEM((B, S_PER, N_OUT), jnp.bfloat16),
            pltpu.VMEM((B, S_PER, N_OUT), jnp.bfloat16),
            pltpu.VMEM((B, S_PER, K), jnp.float32),
            pltpu.VMEM((TILE_M, N_OUT), jnp.bfloat16),
            pltpu.VMEM((TILE_M, N_OUT), jnp.float32),
            pltpu.SemaphoreType.DMA((HOPS * B,)),
            pltpu.SemaphoreType.DMA((HOPS * B,)),
            pltpu.SemaphoreType.DMA((B,)),
            pltpu.SemaphoreType.DMA,
            pltpu.SemaphoreType.DMA,
        ],
        compiler_params=pltpu.CompilerParams(
            vmem_limit_bytes=62 * 1024 * 1024,
        ),
    )(Ob, Wb)
    return out


def kernel(O, Wo):
    b, s_full, h, d = O.shape
    Ob = O.reshape(b, N_DEV, S_PER, h * d)
    Wb = Wo.astype(jnp.bfloat16)
    return _fused_matmul_reduce_scatter(Ob, Wb)


# device time: 1388695 ns/iter; 1.0082x vs baseline; 1.0082x over previous
import jax
import jax.numpy as jnp
from jax import lax
from jax.experimental import pallas as pl
from jax.experimental.pallas import tpu as pltpu

N_DEV = 8
B = 2
S_PER = 512
N_OUT = 8192
K = 1024
TILE_M = 128
N_MTILES = S_PER // TILE_M
HOPS = N_DEV - 1


def _fused_matmul_reduce_scatter(Ob, Wb):

    def body(o_ref, w_ref, out_ref, recv_hbm, w_v, acc_v, vb, o_v, va, vf32,
             send_sems, recv_sems, o_sems, va_sem, st_sem):
        my = lax.axis_index("i")
        right = lax.rem(my + 1, N_DEV)

        def hop_rdma(s, b):
            return pltpu.make_async_remote_copy(
                src_ref=acc_v.at[b],
                dst_ref=recv_hbm.at[s, b],
                send_sem=send_sems.at[s * B + b],
                recv_sem=recv_sems.at[s * B + b],
                device_id=(right,),
                device_id_type=pl.DeviceIdType.MESH,
            )

        def load_o(c):
            for b in range(B):
                pltpu.make_async_copy(
                    o_ref.at[b, c], o_v.at[b], o_sems.at[b]).start()

        def compute_chunk(c, dst):
            for b in range(B):
                pltpu.make_async_copy(
                    o_ref.at[b, c], o_v.at[b], o_sems.at[b]).wait()

                def dot_tile(r, _):
                    sl = pl.ds(r * TILE_M, TILE_M)
                    dst[b, sl] = jnp.dot(
                        o_v[b, sl], w_v[...],
                        preferred_element_type=jnp.float32,
                    ).astype(jnp.bfloat16)
                    return 0

                lax.fori_loop(0, N_MTILES, dot_tile, 0)

        pltpu.make_async_copy(w_ref, w_v, va_sem).start()
        c0 = lax.rem(my + N_DEV - 1, N_DEV)
        load_o(c0)
        pltpu.make_async_copy(w_ref, w_v, va_sem).wait()
        compute_chunk(c0, acc_v)
        for b in range(B):
            hop_rdma(0, b).start()

        def hop(s, _):
            c = lax.rem(my + 2 * N_DEV - 2 - s, N_DEV)
            final = s == HOPS - 1
            load_o(c)
            compute_chunk(c, vb)

            for b in range(B):
                d = hop_rdma(s, b)
                d.wait_recv()
                d.wait_send()

                def add_tile(t, _):
                    sl = pl.ds(t * TILE_M, TILE_M)
                    pltpu.make_async_copy(
                        recv_hbm.at[s, b, sl], va, va_sem).start()
                    pltpu.make_async_copy(
                        recv_hbm.at[s, b, sl], va, va_sem).wait()

                    @pl.when(jnp.logical_not(final))
                    def _():
                        acc_v[b, sl] = va[...] + vb[b, sl]

                    @pl.when(final)
                    def _():
                        vf32[...] = (va[...].astype(jnp.float32)
                                     + vb[b, sl].astype(jnp.float32))
                        pltpu.make_async_copy(
                            vf32, out_ref.at[b, sl], st_sem).start()
                        pltpu.make_async_copy(
                            vf32, out_ref.at[b, sl], st_sem).wait()

                    return 0

                lax.fori_loop(0, N_MTILES, add_tile, 0)

                @pl.when(jnp.logical_not(final))
                def _():
                    hop_rdma(s + 1, b).start()
            return 0

        lax.fori_loop(0, HOPS, hop, 0)

    hbm = pltpu.MemorySpace.HBM
    out, _ = pl.pallas_call(
        body,
        out_shape=(
            jax.ShapeDtypeStruct((B, S_PER, N_OUT), jnp.float32),
            jax.ShapeDtypeStruct((HOPS, B, S_PER, N_OUT), jnp.bfloat16),
        ),
        in_specs=[
            pl.BlockSpec(memory_space=hbm),
            pl.BlockSpec(memory_space=hbm),
        ],
        out_specs=(
            pl.BlockSpec(memory_space=hbm),
            pl.BlockSpec(memory_space=hbm),
        ),
        scratch_shapes=[
            pltpu.VMEM((K, N_OUT), jnp.bfloat16),
            pltpu.VMEM((B, S_PER, N_OUT), jnp.bfloat16),
            pltpu.VMEM((B, S_PER, N_OUT), jnp.bfloat16),
            pltpu.VMEM((B, S_PER, K), jnp.bfloat16),
            pltpu.VMEM((TILE_M, N_OUT), jnp.bfloat16),
            pltpu.VMEM((TILE_M, N_OUT), jnp.float32),
            pltpu.SemaphoreType.DMA((HOPS * B,)),
            pltpu.SemaphoreType.DMA((HOPS * B,)),
            pltpu.SemaphoreType.DMA((B,)),
            pltpu.SemaphoreType.DMA,
            pltpu.SemaphoreType.DMA,
        ],
        compiler_params=pltpu.CompilerParams(
            vmem_limit_bytes=62 * 1024 * 1024,
        ),
    )(Ob, Wb)
    return out


def kernel(O, Wo):
    b, s_full, h, d = O.shape
    Ob = O.reshape(b, N_DEV, S_PER, h * d).astype(jnp.bfloat16)
    Wb = Wo.astype(jnp.bfloat16)
    return _fused_matmul_reduce_scatter(Ob, Wb)
